# baseline (speedup 1.0000x reference)
import jax
import jax.numpy as jnp
from jax.experimental import pallas as pl
from jax.experimental.pallas import tpu as pltpu

_HIDDEN = 4096
_NUM_EXPERTS = 64
_TOP_K = 8
_BLOCK_T = 1024
_CHUNK = 64


def _router_kernel(x_ref, w_ref, logits_ref, topw_ref, topi_ref):
    w = w_ref[...]
    iota = jax.lax.broadcasted_iota(jnp.int32, (_CHUNK, _NUM_EXPERTS), 1)
    iota8 = jax.lax.broadcasted_iota(jnp.int32, (_CHUNK, _TOP_K), 1)
    neg_inf = jnp.float32(-jnp.inf)

    for c in range(_BLOCK_T // _CHUNK):
        rows = slice(c * _CHUNK, (c + 1) * _CHUNK)
        xc = x_ref[rows, :]
        work = jax.lax.dot_general(
            xc, w, (((1,), (1,)), ((), ())), preferred_element_type=jnp.float32
        )
        logits_ref[rows, :] = work
        vacc = jnp.zeros((_CHUNK, _TOP_K), jnp.float32)
        iacc = jnp.zeros((_CHUNK, _TOP_K), jnp.int32)
        for k in range(_TOP_K):
            m = jnp.max(work, axis=1, keepdims=True)
            cand = jnp.where(work == m, iota, _NUM_EXPERTS)
            idx = jnp.min(cand, axis=1, keepdims=True)
            vacc = jnp.where(iota8 == k, m, vacc)
            iacc = jnp.where(iota8 == k, idx, iacc)
            if k < _TOP_K - 1:
                work = jnp.where(cand == idx, neg_inf, work)
        e = jnp.exp(vacc - vacc[:, :1])
        topw_ref[rows, :] = e / jnp.sum(e, axis=1, keepdims=True)
        topi_ref[rows, :] = iacc


@jax.jit
def kernel(hidden_states, gate_w):
    tokens = hidden_states.shape[0]
    grid = (tokens // _BLOCK_T,)
    out_shapes = (
        jax.ShapeDtypeStruct((tokens, _NUM_EXPERTS), jnp.float32),
        jax.ShapeDtypeStruct((tokens, _TOP_K), jnp.float32),
        jax.ShapeDtypeStruct((tokens, _TOP_K), jnp.int32),
    )
    logits, topw, topi = pl.pallas_call(
        _router_kernel,
        grid=grid,
        in_specs=[
            pl.BlockSpec((_BLOCK_T, _HIDDEN), lambda i: (i, 0)),
            pl.BlockSpec((_NUM_EXPERTS, _HIDDEN), lambda i: (0, 0)),
        ],
        out_specs=[
            pl.BlockSpec((_BLOCK_T, _NUM_EXPERTS), lambda i: (i, 0)),
            pl.BlockSpec((_BLOCK_T, _TOP_K), lambda i: (i, 0)),
            pl.BlockSpec((_BLOCK_T, _TOP_K), lambda i: (i, 0)),
        ],
        out_shape=out_shapes,
        compiler_params=pltpu.CompilerParams(
            dimension_semantics=("arbitrary",),
        ),
    )(hidden_states, gate_w)
    return topw, topi, logits


# speedup vs baseline: 1.1461x; 1.1461x over previous
import jax
import jax.numpy as jnp
from jax.experimental import pallas as pl
from jax.experimental.pallas import tpu as pltpu

_HIDDEN = 4096
_NUM_EXPERTS = 64
_TOP_K = 8
_BLOCK_T = 1024
_CHUNK = 128


def _router_kernel(x_ref, w_ref, logits_ref, topw_ref, topi_ref):
    w = w_ref[...]
    iota_f = jax.lax.broadcasted_iota(
        jnp.int32, (_CHUNK, _NUM_EXPERTS), 1
    ).astype(jnp.float32)
    iota8 = jax.lax.broadcasted_iota(jnp.int32, (_CHUNK, _TOP_K), 1)
    neg_inf = jnp.float32(-jnp.inf)
    sentinel = jnp.float32(_NUM_EXPERTS)

    for c in range(_BLOCK_T // _CHUNK):
        rows = slice(c * _CHUNK, (c + 1) * _CHUNK)
        xc = x_ref[rows, :]
        work = jax.lax.dot_general(
            xc, w, (((1,), (1,)), ((), ())), preferred_element_type=jnp.float32
        )
        logits_ref[rows, :] = work
        vacc = jnp.zeros((_CHUNK, _TOP_K), jnp.float32)
        facc = jnp.zeros((_CHUNK, _TOP_K), jnp.float32)
        for k in range(_TOP_K):
            m = jnp.max(work, axis=1, keepdims=True)
            cand = jnp.where(work == m, iota_f, sentinel)
            idx = jnp.min(cand, axis=1, keepdims=True)
            vacc = jnp.where(iota8 == k, m, vacc)
            facc = jnp.where(iota8 == k, idx, facc)
            if k < _TOP_K - 1:
                work = jnp.where(cand == idx, neg_inf, work)
        e = jnp.exp(vacc - vacc[:, :1])
        topw_ref[rows, :] = e / jnp.sum(e, axis=1, keepdims=True)
        topi_ref[rows, :] = facc.astype(jnp.int32)


@jax.jit
def kernel(hidden_states, gate_w):
    tokens = hidden_states.shape[0]
    grid = (tokens // _BLOCK_T,)
    out_shapes = (
        jax.ShapeDtypeStruct((tokens, _NUM_EXPERTS), jnp.float32),
        jax.ShapeDtypeStruct((tokens, _TOP_K), jnp.float32),
        jax.ShapeDtypeStruct((tokens, _TOP_K), jnp.int32),
    )
    logits, topw, topi = pl.pallas_call(
        _router_kernel,
        grid=grid,
        in_specs=[
            pl.BlockSpec((_BLOCK_T, _HIDDEN), lambda i: (i, 0)),
            pl.BlockSpec((_NUM_EXPERTS, _HIDDEN), lambda i: (0, 0)),
        ],
        out_specs=[
            pl.BlockSpec((_BLOCK_T, _NUM_EXPERTS), lambda i: (i, 0)),
            pl.BlockSpec((_BLOCK_T, _TOP_K), lambda i: (i, 0)),
            pl.BlockSpec((_BLOCK_T, _TOP_K), lambda i: (i, 0)),
        ],
        out_shape=out_shapes,
        compiler_params=pltpu.CompilerParams(
            dimension_semantics=("arbitrary",),
        ),
    )(hidden_states, gate_w)
    return topw, topi, logits
